# baseline (speedup 1.0000x reference)
import functools

import jax
import jax.numpy as jnp
from jax import lax
from jax.experimental import pallas as pl
from jax.experimental.pallas import tpu as pltpu
from jax.experimental.pallas import tpu_sc as plsc


def _sc_gather(P: int, D: int, dtype, table):
    info = plsc.get_sparse_core_info()
    NC, NS = info.num_cores, info.num_subcores
    NW = NC * NS
    rows_per_w = P // NW
    mesh = plsc.VectorSubcoreMesh(core_axis_name="c", subcore_axis_name="s")

    @functools.partial(
        pl.kernel,
        mesh=mesh,
        out_type=jax.ShapeDtypeStruct((P, D), dtype),
    )
    def gather_kernel(table_hbm, out_hbm):
        wid = lax.axis_index("s") * NC + lax.axis_index("c")
        base = wid * rows_per_w
        pltpu.sync_copy(
            table_hbm.at[pl.ds(base, rows_per_w), :],
            out_hbm.at[pl.ds(base, rows_per_w), :],
        )

    return gather_kernel(table)


def _tc_broadcast(B: int, P: int, D: int, dtype, rows):
    def body(emb_ref, out_ref):
        out_ref[0] = emb_ref[...]

    return pl.pallas_call(
        body,
        grid=(B,),
        in_specs=[pl.BlockSpec((P, D), lambda b: (0, 0))],
        out_specs=pl.BlockSpec((1, P, D), lambda b: (b, 0, 0)),
        out_shape=jax.ShapeDtypeStruct((B, P, D), dtype),
        compiler_params=pltpu.CompilerParams(
            dimension_semantics=("arbitrary",),
        ),
    )(rows)


def kernel(x, pos_embed):
    B, C, H, W = x.shape
    P = H * W
    D = pos_embed.shape[1]
    rows = _sc_gather(P, D, pos_embed.dtype, pos_embed)
    return _tc_broadcast(B, P, D, pos_embed.dtype, rows)


# speedup vs baseline: 3.3564x; 3.3564x over previous
import functools

import jax
import jax.numpy as jnp
from jax import lax
from jax.experimental import pallas as pl
from jax.experimental.pallas import tpu as pltpu
from jax.experimental.pallas import tpu_sc as plsc


def _sc_gather(P: int, D: int, dtype, table):
    info = plsc.get_sparse_core_info()
    NC, NS = info.num_cores, info.num_subcores
    NW = NC * NS
    rows_per_w = P // NW
    mesh = plsc.VectorSubcoreMesh(core_axis_name="c", subcore_axis_name="s")

    @functools.partial(
        pl.kernel,
        mesh=mesh,
        out_type=jax.ShapeDtypeStruct((P, D), dtype),
        scratch_types=[pltpu.VMEM((rows_per_w, D), dtype)],
    )
    def gather_kernel(table_hbm, out_hbm, chunk_v):
        wid = lax.axis_index("s") * NC + lax.axis_index("c")
        base = wid * rows_per_w
        pltpu.sync_copy(table_hbm.at[pl.ds(base, rows_per_w), :], chunk_v)
        pltpu.sync_copy(chunk_v, out_hbm.at[pl.ds(base, rows_per_w), :])

    return gather_kernel(table)


def _tc_broadcast(B: int, P: int, D: int, dtype, rows):
    def body(emb_ref, out_ref):
        out_ref[0] = emb_ref[...]

    return pl.pallas_call(
        body,
        grid=(B,),
        in_specs=[pl.BlockSpec((P, D), lambda b: (0, 0))],
        out_specs=pl.BlockSpec((1, P, D), lambda b: (b, 0, 0)),
        out_shape=jax.ShapeDtypeStruct((B, P, D), dtype),
        compiler_params=pltpu.CompilerParams(
            dimension_semantics=("arbitrary",),
        ),
    )(rows)


def kernel(x, pos_embed):
    B, C, H, W = x.shape
    P = H * W
    D = pos_embed.shape[1]
    rows = _sc_gather(P, D, pos_embed.dtype, pos_embed)
    return _tc_broadcast(B, P, D, pos_embed.dtype, rows)


# speedup vs baseline: 7.4439x; 2.2179x over previous
import jax
import jax.numpy as jnp
from jax.experimental import pallas as pl
from jax.experimental.pallas import tpu as pltpu

_CHUNKS = 8


def _tc_broadcast(B: int, P: int, D: int, dtype, table):
    nch = _CHUNKS if P % _CHUNKS == 0 else 1
    rows_c = P // nch

    def body(emb_hbm, out_hbm, rows_vmem, in_sems, out_sem):
        reads = [
            pltpu.make_async_copy(
                emb_hbm.at[pl.ds(i * rows_c, rows_c), :],
                rows_vmem.at[pl.ds(i * rows_c, rows_c), :],
                in_sems.at[i],
            )
            for i in range(nch)
        ]
        for r in reads:
            r.start()
        writes = []
        for i in range(nch):
            reads[i].wait()
            for b in range(B):
                cp = pltpu.make_async_copy(
                    rows_vmem.at[pl.ds(i * rows_c, rows_c), :],
                    out_hbm.at[b, pl.ds(i * rows_c, rows_c), :],
                    out_sem,
                )
                cp.start()
                writes.append(cp)
        for cp in writes:
            cp.wait()

    return pl.pallas_call(
        body,
        in_specs=[pl.BlockSpec(memory_space=pl.ANY)],
        out_specs=pl.BlockSpec(memory_space=pl.ANY),
        out_shape=jax.ShapeDtypeStruct((B, P, D), dtype),
        scratch_shapes=[
            pltpu.VMEM((P, D), dtype),
            pltpu.SemaphoreType.DMA((nch,)),
            pltpu.SemaphoreType.DMA,
        ],
    )(table)


def kernel(x, pos_embed):
    B, C, H, W = x.shape
    P = H * W
    D = pos_embed.shape[1]
    return _tc_broadcast(B, P, D, pos_embed.dtype, pos_embed)
